# baseline (speedup 1.0000x reference)
import functools

import jax
import jax.numpy as jnp
from jax import lax
from jax.experimental import pallas as pl
from jax.experimental.pallas import tpu as pltpu
from jax.experimental.pallas import tpu_sc as plsc

N = 262144
C = 20
K = 1024
NC, NS, L = 2, 16, 16
NW = NC * NS
ROWS_PER_TILE = N // NW
R = 2048
NCHUNK = ROWS_PER_TILE // R


def _histo_body(preds_hbm, labels_hbm, out_hbm, pbuf, lbuf, hist):
    wid = lax.axis_index("s") * NC + lax.axis_index("c")

    def zero_body(i, carry):
        hist[pl.ds(i * L, L)] = jnp.zeros((L,), jnp.int32)
        return carry

    lax.fori_loop(0, C * K // L, zero_body, 0)

    lane = lax.iota(jnp.int32, L)
    cvecs, drvecs = [], []
    for v in range(5):
        flat = lane + 16 * v
        cvecs.append(flat % C)
        drvecs.append(flat // C)
    idx_bases = [cv * K for cv in cvecs]
    one_f = jnp.full((L,), 1.0, jnp.float32)
    kf = jnp.full((L,), float(K) - 0.25, jnp.float32)
    v_fg = jnp.full((L,), 65537, jnp.int32)
    v_bg = jnp.full((L,), 1, jnp.int32)

    row_base = wid * ROWS_PER_TILE

    def chunk_body(k, carry):
        base = row_base + k * R
        pltpu.sync_copy(preds_hbm.at[pl.ds(base, R)], pbuf)
        pltpu.sync_copy(labels_hbm.at[pl.ds(base, R)], lbuf)

        def group_body(g, gcarry):
            for u in range(2):
                gg = 2 * g + u
                r0 = jnp.full((L,), 4 * gg, jnp.int32)
                for v in range(5):
                    rows = r0 + drvecs[v]
                    p = plsc.load_gather(pbuf, [rows, cvecs[v]])
                    lbl = plsc.load_gather(lbuf, [rows])
                    eq = lbl == cvecs[v]
                    e = jnp.where(eq, one_f - p, p)
                    b = (e * kf).astype(jnp.int32)
                    plsc.addupdate_scatter(hist, [idx_bases[v] + b],
                                           jnp.where(eq, v_fg, v_bg))
            return gcarry

        lax.fori_loop(0, R // 8, group_body, 0)
        return carry

    lax.fori_loop(0, NCHUNK, chunk_body, 0)
    pltpu.sync_copy(hist, out_hbm.at[wid])


@functools.cache
def _histo():
    return pl.kernel(
        _histo_body,
        out_type=jax.ShapeDtypeStruct((NW, C * K), jnp.int32),
        mesh=plsc.VectorSubcoreMesh(core_axis_name="c", subcore_axis_name="s",
                                    num_cores=NC, num_subcores=NS),
        compiler_params=pltpu.CompilerParams(
            use_tc_tiling_on_sc=False, needs_layout_passes=False),
        scratch_types=[
            pltpu.VMEM((R, C), jnp.float32),
            pltpu.VMEM((R,), jnp.int32),
            pltpu.VMEM((C * K,), jnp.int32),
        ],
    )


def _finish_body(hist_ref, out_ref):
    v = hist_ref[...]
    n = jnp.sum(v & 0xFFFF, axis=0).astype(jnp.float32)
    m = jnp.sum(v >> 16, axis=0).astype(jnp.float32)
    row = lax.broadcasted_iota(jnp.int32, (K, K), 0)
    col = lax.broadcasted_iota(jnp.int32, (K, K), 1)
    tri = (row >= col).astype(jnp.float32)
    cum_n = jnp.dot(n, tri, preferred_element_type=jnp.float32)
    cum_m = jnp.dot(m, tri, preferred_element_type=jnp.float32)
    p_tot = cum_m[:, 0:1]
    union = p_tot + cum_n - cum_m
    jac = jnp.where(cum_n > 0.0,
                    1.0 - (p_tot - cum_m) / jnp.maximum(union, 1.0),
                    0.0)
    bin_id = lax.broadcasted_iota(jnp.int32, (C, K), 1)
    w = jnp.where(bin_id == 0, 0.5 / K, 1.0 / K)
    losses = jnp.sum(jac * w, axis=1)
    present = (p_tot[:, 0] > 0.0).astype(jnp.float32)
    result = jnp.sum(losses * present) / jnp.maximum(jnp.sum(present), 1.0)
    out_ref[...] = jnp.reshape(result, (1, 1))


_finish = pl.pallas_call(
    _finish_body,
    out_shape=jax.ShapeDtypeStruct((1, 1), jnp.float32),
)


def kernel(preds, labels):
    hist = _histo()(preds, labels.astype(jnp.int32))
    out = _finish(hist.reshape(NW, C, K))
    return out.reshape(())


# speedup vs baseline: 1.3137x; 1.3137x over previous
import functools

import jax
import jax.numpy as jnp
from jax import lax
from jax.experimental import pallas as pl
from jax.experimental.pallas import tpu as pltpu
from jax.experimental.pallas import tpu_sc as plsc

N = 262144
C = 20
K = 1024
NC, NS, L = 2, 16, 16
NW = NC * NS
ROWS_PER_TILE = N // NW
R = 2048
NCHUNK = ROWS_PER_TILE // R


def _histo_body(preds_hbm, labels_hbm, out_hbm, pbuf, lbuf, hist):
    wid = lax.axis_index("s") * NC + lax.axis_index("c")

    def zero_body(i, carry):
        hist[pl.ds(i * L, L)] = jnp.zeros((L,), jnp.int32)
        return carry

    lax.fori_loop(0, C * K // L, zero_body, 0)

    lane = lax.iota(jnp.int32, L)
    cvecs, drvecs = [], []
    for v in range(5):
        flat = lane + 16 * v
        cvecs.append(flat % C)
        drvecs.append(flat // C)
    idx_bases = [cv * K for cv in cvecs]
    one_f = jnp.full((L,), 1.0, jnp.float32)
    kf = jnp.full((L,), float(K) - 0.25, jnp.float32)
    v_fg = jnp.full((L,), 65537, jnp.int32)
    v_bg = jnp.full((L,), 1, jnp.int32)

    row_base = wid * ROWS_PER_TILE

    def chunk_body(k, carry):
        base = row_base + k * R
        pltpu.sync_copy(preds_hbm.at[pl.ds(base, R)], pbuf)
        pltpu.sync_copy(labels_hbm.at[pl.ds(base, R)], lbuf)

        def group_body(g):
            r0 = jnp.full((L,), 4 * g, jnp.int32)
            for v in range(5):
                rows = r0 + drvecs[v]
                p = plsc.load_gather(pbuf, [rows, cvecs[v]])
                lbl = plsc.load_gather(lbuf, [rows])
                eq = lbl == cvecs[v]
                e = jnp.where(eq, one_f - p, p)
                b = (e * kf).astype(jnp.int32)
                plsc.addupdate_scatter(hist, [idx_bases[v] + b],
                                       jnp.where(eq, v_fg, v_bg))

        plsc.parallel_loop(0, R // 4, 1, unroll=4)(group_body)
        return carry

    lax.fori_loop(0, NCHUNK, chunk_body, 0)
    pltpu.sync_copy(hist, out_hbm.at[wid])


@functools.cache
def _histo():
    return pl.kernel(
        _histo_body,
        out_type=jax.ShapeDtypeStruct((NW, C * K), jnp.int32),
        mesh=plsc.VectorSubcoreMesh(core_axis_name="c", subcore_axis_name="s",
                                    num_cores=NC, num_subcores=NS),
        compiler_params=pltpu.CompilerParams(
            use_tc_tiling_on_sc=False, needs_layout_passes=False),
        scratch_types=[
            pltpu.VMEM((R, C), jnp.float32),
            pltpu.VMEM((R,), jnp.int32),
            pltpu.VMEM((C * K,), jnp.int32),
        ],
    )


def _finish_body(hist_ref, out_ref):
    v = hist_ref[...]
    n = jnp.sum(v & 0xFFFF, axis=0).astype(jnp.float32)
    m = jnp.sum(v >> 16, axis=0).astype(jnp.float32)
    row = lax.broadcasted_iota(jnp.int32, (K, K), 0)
    col = lax.broadcasted_iota(jnp.int32, (K, K), 1)
    tri = (row >= col).astype(jnp.float32)
    cum_n = jnp.dot(n, tri, preferred_element_type=jnp.float32)
    cum_m = jnp.dot(m, tri, preferred_element_type=jnp.float32)
    p_tot = cum_m[:, 0:1]
    union = p_tot + cum_n - cum_m
    jac = jnp.where(cum_n > 0.0,
                    1.0 - (p_tot - cum_m) / jnp.maximum(union, 1.0),
                    0.0)
    bin_id = lax.broadcasted_iota(jnp.int32, (C, K), 1)
    w = jnp.where(bin_id == 0, 0.5 / K, 1.0 / K)
    losses = jnp.sum(jac * w, axis=1)
    present = (p_tot[:, 0] > 0.0).astype(jnp.float32)
    result = jnp.sum(losses * present) / jnp.maximum(jnp.sum(present), 1.0)
    out_ref[...] = jnp.reshape(result, (1, 1))


_finish = pl.pallas_call(
    _finish_body,
    out_shape=jax.ShapeDtypeStruct((1, 1), jnp.float32),
)


def kernel(preds, labels):
    hist = _histo()(preds, labels.astype(jnp.int32))
    out = _finish(hist.reshape(NW, C, K))
    return out.reshape(())


# speedup vs baseline: 1.7145x; 1.3051x over previous
import functools

import jax
import jax.numpy as jnp
from jax import lax
from jax.experimental import pallas as pl
from jax.experimental.pallas import tpu as pltpu
from jax.experimental.pallas import tpu_sc as plsc

N = 262144
C = 20
K = 1024
NC, NS, L = 2, 16, 16
NW = NC * NS
ROWS_PER_TILE = N // NW
R = 2048
NCHUNK = ROWS_PER_TILE // R


def _histo_body(preds_hbm, labels_hbm, out_hbm, pbuf, lbuf, hist):
    wid = lax.axis_index("s") * NC + lax.axis_index("c")

    def zero_body(i, carry):
        hist[pl.ds(i * L, L)] = jnp.zeros((L,), jnp.int32)
        return carry

    lax.fori_loop(0, C * K // L, zero_body, 0)

    lane = lax.iota(jnp.int32, L)
    cvecs, drvecs = [], []
    for v in range(5):
        flat = lane + 16 * v
        cvecs.append(flat % C)
        drvecs.append(flat // C)
    idx_bases = [cv * K for cv in cvecs]
    one_f = jnp.full((L,), 1.0, jnp.float32)
    kf = jnp.full((L,), float(K) - 0.25, jnp.float32)
    v_fg = jnp.full((L,), 65537, jnp.int32)
    v_bg = jnp.full((L,), 1, jnp.int32)

    row_base = wid * ROWS_PER_TILE

    def chunk_body(k, carry):
        base = row_base + k * R
        pltpu.sync_copy(preds_hbm.at[pl.ds(base * C, R * C)], pbuf)
        pltpu.sync_copy(labels_hbm.at[pl.ds(base, R)], lbuf)

        def group_body(g):
            r0 = jnp.full((L,), 4 * g, jnp.int32)
            fbase = g * (4 * C)
            for v in range(5):
                p = pbuf[pl.ds(fbase + 16 * v, L)]
                lbl = plsc.load_gather(lbuf, [r0 + drvecs[v]])
                eq = lbl == cvecs[v]
                e = jnp.where(eq, one_f - p, p)
                b = (e * kf).astype(jnp.int32)
                plsc.addupdate_scatter(hist, [idx_bases[v] + b],
                                       jnp.where(eq, v_fg, v_bg))

        plsc.parallel_loop(0, R // 4, 1, unroll=4)(group_body)
        return carry

    lax.fori_loop(0, NCHUNK, chunk_body, 0)
    pltpu.sync_copy(hist, out_hbm.at[wid])


@functools.cache
def _histo():
    return pl.kernel(
        _histo_body,
        out_type=jax.ShapeDtypeStruct((NW, C * K), jnp.int32),
        mesh=plsc.VectorSubcoreMesh(core_axis_name="c", subcore_axis_name="s",
                                    num_cores=NC, num_subcores=NS),
        compiler_params=pltpu.CompilerParams(
            use_tc_tiling_on_sc=False, needs_layout_passes=False),
        scratch_types=[
            pltpu.VMEM((R * C,), jnp.float32),
            pltpu.VMEM((R,), jnp.int32),
            pltpu.VMEM((C * K,), jnp.int32),
        ],
    )


def _finish_body(hist_ref, out_ref):
    v = hist_ref[...]
    n = jnp.sum(v & 0xFFFF, axis=0).astype(jnp.float32)
    m = jnp.sum(v >> 16, axis=0).astype(jnp.float32)
    row = lax.broadcasted_iota(jnp.int32, (K, K), 0)
    col = lax.broadcasted_iota(jnp.int32, (K, K), 1)
    tri = (row >= col).astype(jnp.float32)
    cum_n = jnp.dot(n, tri, preferred_element_type=jnp.float32)
    cum_m = jnp.dot(m, tri, preferred_element_type=jnp.float32)
    p_tot = cum_m[:, 0:1]
    union = p_tot + cum_n - cum_m
    jac = jnp.where(cum_n > 0.0,
                    1.0 - (p_tot - cum_m) / jnp.maximum(union, 1.0),
                    0.0)
    bin_id = lax.broadcasted_iota(jnp.int32, (C, K), 1)
    w = jnp.where(bin_id == 0, 0.5 / K, 1.0 / K)
    losses = jnp.sum(jac * w, axis=1)
    present = (p_tot[:, 0] > 0.0).astype(jnp.float32)
    result = jnp.sum(losses * present) / jnp.maximum(jnp.sum(present), 1.0)
    out_ref[...] = jnp.reshape(result, (1, 1))


_finish = pl.pallas_call(
    _finish_body,
    out_shape=jax.ShapeDtypeStruct((1, 1), jnp.float32),
)


def kernel(preds, labels):
    hist = _histo()(preds.reshape(-1), labels)
    out = _finish(hist.reshape(NW, C, K))
    return out.reshape(())
